# baseline (speedup 1.0000x reference)
import functools

import jax
import jax.numpy as jnp
from jax.experimental import pallas as pl
from jax.experimental.pallas import tpu as pltpu

N = 8192
D = 2048
K4 = 4
RB = 512
MB, NB = 256, 512


def _merge_top4(acc, d2):
    work = jnp.concatenate([acc, d2], axis=1)
    outs = []
    for t in range(K4):
        m = jnp.min(work, axis=1, keepdims=True)
        outs.append(m)
        if t < K4 - 1:
            work = jnp.where(work == m, jnp.inf, work)
    return jnp.concatenate(outs, axis=1)


def _col_top4(d2):
    work = d2
    outs = []
    for t in range(K4):
        m = jnp.min(work, axis=0, keepdims=True)
        outs.append(m)
        if t < K4 - 1:
            work = jnp.where(work == m, jnp.inf, work)
    return jnp.transpose(jnp.concatenate(outs, axis=0))


def _norms_body(x_ref, out_ref):
    x = x_ref[...]
    out_ref[...] = jnp.sum(x * x, axis=1, keepdims=True)


def _sq_norms(x):
    b = min(1024, N)
    return pl.pallas_call(
        _norms_body,
        grid=(N // b,),
        in_specs=[pl.BlockSpec((b, D), lambda i: (i, 0))],
        out_specs=pl.BlockSpec((b, 1), lambda i: (i, 0)),
        out_shape=jax.ShapeDtypeStruct((N, 1), jnp.float32),
    )(x)


def _radii_tri_body(ii_ref, jj_ref, xi_ref, xj_ref, ni_ref, nj_ref,
                    out_ref, acc_ref):
    t = pl.program_id(0)
    nt = pl.num_programs(0)
    b = xi_ref.shape[0]
    ii = ii_ref[t]
    jj = jj_ref[t]

    @pl.when(t == 0)
    def _init():
        acc_ref[...] = jnp.full_like(acc_ref, jnp.inf)

    xi = xi_ref[...]
    xj = xj_ref[...]
    g = jax.lax.dot_general(xi, xj, (((1,), (1,)), ((), ())),
                            preferred_element_type=jnp.float32)
    d2 = ni_ref[...] + (nj_ref[...] - 2.0 * g)

    acc_ref[pl.ds(ii * b, b), :] = _merge_top4(acc_ref[pl.ds(ii * b, b), :],
                                               d2)

    @pl.when(ii != jj)
    def _col():
        acc_ref[pl.ds(jj * b, b), :] = _merge_top4(
            acc_ref[pl.ds(jj * b, b), :], _col_top4(d2))

    @pl.when(t == nt - 1)
    def _emit():
        out_ref[...] = jnp.maximum(acc_ref[:, K4 - 1:K4], 0.0)


def _radii2(x, norms_col, norms_row, b):
    nb = N // b
    pairs = [(i, j) for i in range(nb) for j in range(i, nb)]
    ii = jnp.asarray([p[0] for p in pairs], dtype=jnp.int32)
    jj = jnp.asarray([p[1] for p in pairs], dtype=jnp.int32)
    grid_spec = pltpu.PrefetchScalarGridSpec(
        num_scalar_prefetch=2,
        grid=(len(pairs),),
        in_specs=[
            pl.BlockSpec((b, D), lambda t, ii, jj: (ii[t], 0)),
            pl.BlockSpec((b, D), lambda t, ii, jj: (jj[t], 0)),
            pl.BlockSpec((b, 1), lambda t, ii, jj: (ii[t], 0)),
            pl.BlockSpec((1, b), lambda t, ii, jj: (0, jj[t])),
        ],
        out_specs=pl.BlockSpec((N, 1), lambda t, ii, jj: (0, 0)),
        scratch_shapes=[pltpu.VMEM((N, K4), jnp.float32)],
    )
    return pl.pallas_call(
        _radii_tri_body,
        grid_spec=grid_spec,
        out_shape=jax.ShapeDtypeStruct((N, 1), jnp.float32),
        compiler_params=pltpu.CompilerParams(
            dimension_semantics=("arbitrary",),
        ),
    )(ii, jj, x, x, norms_col, norms_row)


def _mask_body(f_ref, r_ref, nf_ref, nr_ref, rr_ref, rf_ref, out_ref,
               p_scr, r_scr):
    i = pl.program_id(0)
    j = pl.program_id(1)
    ni_ = pl.num_programs(0)
    nj_ = pl.num_programs(1)
    bm = f_ref.shape[0]
    bn = r_ref.shape[0]

    @pl.when((i == 0) & (j == 0))
    def _init():
        p_scr[...] = jnp.zeros_like(p_scr)
        r_scr[...] = jnp.zeros_like(r_scr)

    f = f_ref[...]
    r = r_ref[...]
    g = jax.lax.dot_general(f, r, (((1,), (1,)), ((), ())),
                            preferred_element_type=jnp.float32)
    d2 = nf_ref[...] + (nr_ref[...] - 2.0 * g)

    hit_p = jnp.max((d2 <= rr_ref[...]).astype(jnp.float32), axis=1,
                    keepdims=True)
    p_scr[pl.ds(i * bm, bm), :] = jnp.maximum(p_scr[pl.ds(i * bm, bm), :],
                                              hit_p)
    hit_r = jnp.max((d2 <= rf_ref[...]).astype(jnp.float32), axis=0,
                    keepdims=True)
    r_scr[:, pl.ds(j * bn, bn)] = jnp.maximum(r_scr[:, pl.ds(j * bn, bn)],
                                              hit_r)

    @pl.when((i == ni_ - 1) & (j == nj_ - 1))
    def _emit():
        out_ref[0, 0] = jnp.sum(p_scr[...]) * (1.0 / N)
        out_ref[0, 1] = jnp.sum(r_scr[...]) * (1.0 / N)


def _masks(fake, real, nf_col, nr_row, radii2_real_row, radii2_fake_col,
           bm, bn):
    grid = (N // bm, N // bn)
    return pl.pallas_call(
        _mask_body,
        grid=grid,
        in_specs=[
            pl.BlockSpec((bm, D), lambda i, j: (i, 0)),
            pl.BlockSpec((bn, D), lambda i, j: (j, 0)),
            pl.BlockSpec((bm, 1), lambda i, j: (i, 0)),
            pl.BlockSpec((1, bn), lambda i, j: (0, j)),
            pl.BlockSpec((1, bn), lambda i, j: (0, j)),
            pl.BlockSpec((bm, 1), lambda i, j: (i, 0)),
        ],
        out_specs=pl.BlockSpec(memory_space=pltpu.SMEM),
        out_shape=jax.ShapeDtypeStruct((1, 2), jnp.float32),
        scratch_shapes=[
            pltpu.VMEM((N, 1), jnp.float32),
            pltpu.VMEM((1, N), jnp.float32),
        ],
        compiler_params=pltpu.CompilerParams(
            dimension_semantics=("arbitrary", "arbitrary"),
        ),
    )(fake, real, nf_col, nr_row, radii2_real_row, radii2_fake_col)


@functools.partial(jax.jit, static_argnames=())
def kernel(real_feats, fake_feats):
    nr = _sq_norms(real_feats)
    nf = _sq_norms(fake_feats)
    nr_row = nr.reshape(1, N)
    nf_row = nf.reshape(1, N)
    radii2_real = _radii2(real_feats, nr, nr_row, RB)
    radii2_fake = _radii2(fake_feats, nf, nf_row, RB)
    out = _masks(fake_feats, real_feats, nf, nr_row,
                 radii2_real.reshape(1, N), radii2_fake, MB, NB)
    return out.reshape(2)


# speedup vs baseline: 1.0704x; 1.0704x over previous
import functools

import jax
import jax.numpy as jnp
from jax.experimental import pallas as pl
from jax.experimental.pallas import tpu as pltpu

N = 8192
D = 2048
K4 = 4
RB = 512
MB, NB = 256, 512


def _merge_top4(acc, d2):
    work = jnp.concatenate([acc, d2], axis=1)
    outs = []
    for t in range(K4):
        m = jnp.min(work, axis=1, keepdims=True)
        outs.append(m)
        if t < K4 - 1:
            work = jnp.where(work == m, jnp.inf, work)
    return jnp.concatenate(outs, axis=1)


def _col_top4(d2):
    work = d2
    outs = []
    for t in range(K4):
        m = jnp.min(work, axis=0, keepdims=True)
        outs.append(m)
        if t < K4 - 1:
            work = jnp.where(work == m, jnp.inf, work)
    return jnp.transpose(jnp.concatenate(outs, axis=0))


def _norms_body(x_ref, out_ref):
    x = x_ref[...]
    out_ref[...] = jnp.sum(x * x, axis=1, keepdims=True)


def _sq_norms(x):
    b = min(1024, N)
    return pl.pallas_call(
        _norms_body,
        grid=(N // b,),
        in_specs=[pl.BlockSpec((b, D), lambda i: (i, 0))],
        out_specs=pl.BlockSpec((b, 1), lambda i: (i, 0)),
        out_shape=jax.ShapeDtypeStruct((N, 1), jnp.float32),
    )(x)


def _radii_tri_body(ii_ref, jj_ref, xi_ref, xj_ref, ni_ref, nj_ref,
                    out_ref, acc_ref):
    t = pl.program_id(0)
    nt = pl.num_programs(0)
    b = xi_ref.shape[0]
    ii = ii_ref[t]
    jj = jj_ref[t]

    @pl.when(t == 0)
    def _init():
        acc_ref[...] = jnp.full_like(acc_ref, jnp.inf)

    xi = xi_ref[...]
    xj = xj_ref[...]
    g = jax.lax.dot_general(xi, xj, (((1,), (1,)), ((), ())),
                            preferred_element_type=jnp.float32)
    d2 = ni_ref[...] + (nj_ref[...] - 2.0 * g)

    acc_ref[pl.ds(ii * b, b), :] = _merge_top4(acc_ref[pl.ds(ii * b, b), :],
                                               d2)

    @pl.when(ii != jj)
    def _col():
        acc_ref[pl.ds(jj * b, b), :] = _merge_top4(
            acc_ref[pl.ds(jj * b, b), :], _col_top4(d2))

    @pl.when(t == nt - 1)
    def _emit():
        out_ref[...] = jnp.maximum(acc_ref[:, K4 - 1:K4], 0.0)


def _radii2(x, norms_col, norms_row, b):
    nb = N // b
    pairs = [(i, j) for i in range(nb) for j in range(i, nb)]
    ii = jnp.asarray([p[0] for p in pairs], dtype=jnp.int32)
    jj = jnp.asarray([p[1] for p in pairs], dtype=jnp.int32)
    grid_spec = pltpu.PrefetchScalarGridSpec(
        num_scalar_prefetch=2,
        grid=(len(pairs),),
        in_specs=[
            pl.BlockSpec((b, D), lambda t, ii, jj: (ii[t], 0)),
            pl.BlockSpec((b, D), lambda t, ii, jj: (jj[t], 0)),
            pl.BlockSpec((b, 1), lambda t, ii, jj: (ii[t], 0)),
            pl.BlockSpec((1, b), lambda t, ii, jj: (0, jj[t])),
        ],
        out_specs=pl.BlockSpec((N, 1), lambda t, ii, jj: (0, 0)),
        scratch_shapes=[pltpu.VMEM((N, K4), jnp.float32)],
    )
    return pl.pallas_call(
        _radii_tri_body,
        grid_spec=grid_spec,
        out_shape=jax.ShapeDtypeStruct((N, 1), jnp.float32),
        compiler_params=pltpu.CompilerParams(
            dimension_semantics=("arbitrary",),
        ),
    )(ii, jj, x, x, norms_col, norms_row)


def _mask_body(f_ref, r_ref, nf_ref, nr_ref, rr_ref, rf_ref, out_ref,
               p_scr, r_scr):
    i = pl.program_id(0)
    j = pl.program_id(1)
    ni_ = pl.num_programs(0)
    nj_ = pl.num_programs(1)
    bm = f_ref.shape[0]
    bn = r_ref.shape[0]

    @pl.when((i == 0) & (j == 0))
    def _init():
        p_scr[...] = jnp.zeros_like(p_scr)
        r_scr[...] = jnp.zeros_like(r_scr)

    f = f_ref[...]
    r = r_ref[...]
    g = jax.lax.dot_general(f, r, (((1,), (1,)), ((), ())),
                            preferred_element_type=jnp.float32)
    d2 = nf_ref[...] + (nr_ref[...] - 2.0 * g)

    hit_p = jnp.max((d2 <= rr_ref[...]).astype(jnp.float32), axis=1,
                    keepdims=True)
    p_scr[pl.ds(i * bm, bm), :] = jnp.maximum(p_scr[pl.ds(i * bm, bm), :],
                                              hit_p)
    hit_r = jnp.max((d2 <= rf_ref[...]).astype(jnp.float32), axis=0,
                    keepdims=True)
    r_scr[:, pl.ds(j * bn, bn)] = jnp.maximum(r_scr[:, pl.ds(j * bn, bn)],
                                              hit_r)

    @pl.when((i == ni_ - 1) & (j == nj_ - 1))
    def _emit():
        out_ref[0, 0] = jnp.sum(p_scr[...]) * (1.0 / N)
        out_ref[0, 1] = jnp.sum(r_scr[...]) * (1.0 / N)


def _masks(fake, real, nf_col, nr_row, radii2_real_row, radii2_fake_col,
           bm, bn):
    grid = (N // bm, N // bn)
    return pl.pallas_call(
        _mask_body,
        grid=grid,
        in_specs=[
            pl.BlockSpec((bm, D), lambda i, j: (i, 0)),
            pl.BlockSpec((bn, D), lambda i, j: (j, 0)),
            pl.BlockSpec((bm, 1), lambda i, j: (i, 0)),
            pl.BlockSpec((1, bn), lambda i, j: (0, j)),
            pl.BlockSpec((1, bn), lambda i, j: (0, j)),
            pl.BlockSpec((bm, 1), lambda i, j: (i, 0)),
        ],
        out_specs=pl.BlockSpec(memory_space=pltpu.SMEM),
        out_shape=jax.ShapeDtypeStruct((1, 2), jnp.float32),
        scratch_shapes=[
            pltpu.VMEM((N, 1), jnp.float32),
            pltpu.VMEM((1, N), jnp.float32),
        ],
        compiler_params=pltpu.CompilerParams(
            dimension_semantics=("arbitrary", "arbitrary"),
        ),
    )(fake, real, nf_col, nr_row, radii2_real_row, radii2_fake_col)


@functools.partial(jax.jit, static_argnames=())
def kernel(real_feats, fake_feats):
    nr = _sq_norms(real_feats)
    nf = _sq_norms(fake_feats)
    nr_row = nr.reshape(1, N)
    nf_row = nf.reshape(1, N)
    real_b = real_feats.astype(jnp.bfloat16)
    fake_b = fake_feats.astype(jnp.bfloat16)
    radii2_real = _radii2(real_b, nr, nr_row, RB)
    radii2_fake = _radii2(fake_b, nf, nf_row, RB)
    out = _masks(fake_b, real_b, nf, nr_row,
                 radii2_real.reshape(1, N), radii2_fake, MB, NB)
    return out.reshape(2)


# speedup vs baseline: 1.5043x; 1.4054x over previous
import functools

import jax
import jax.numpy as jnp
from jax.experimental import pallas as pl
from jax.experimental.pallas import tpu as pltpu

N = 8192
D = 2048
K4 = 4
RB = 1024
MB, NB = 512, 512


def _merge_top4(acc, d2):
    work = jnp.concatenate([acc, d2], axis=1)
    outs = []
    for t in range(K4):
        m = jnp.min(work, axis=1, keepdims=True)
        outs.append(m)
        if t < K4 - 1:
            work = jnp.where(work == m, jnp.inf, work)
    return jnp.concatenate(outs, axis=1)


def _col_top4(d2):
    work = d2
    outs = []
    for t in range(K4):
        m = jnp.min(work, axis=0, keepdims=True)
        outs.append(m)
        if t < K4 - 1:
            work = jnp.where(work == m, jnp.inf, work)
    return jnp.transpose(jnp.concatenate(outs, axis=0))


def _norms_body(x_ref, out_ref):
    x = x_ref[...]
    out_ref[...] = jnp.sum(x * x, axis=1, keepdims=True)


def _sq_norms(x):
    b = min(1024, N)
    return pl.pallas_call(
        _norms_body,
        grid=(N // b,),
        in_specs=[pl.BlockSpec((b, D), lambda i: (i, 0))],
        out_specs=pl.BlockSpec((b, 1), lambda i: (i, 0)),
        out_shape=jax.ShapeDtypeStruct((N, 1), jnp.float32),
    )(x)


def _radii_tri_body(ii_ref, jj_ref, xi_ref, xj_ref, ni_ref, nj_ref,
                    out_ref, acc_ref):
    t = pl.program_id(0)
    nt = pl.num_programs(0)
    b = xi_ref.shape[0]
    ii = ii_ref[t]
    jj = jj_ref[t]

    @pl.when(t == 0)
    def _init():
        acc_ref[...] = jnp.full_like(acc_ref, jnp.inf)

    xi = xi_ref[...]
    xj = xj_ref[...]
    g = jax.lax.dot_general(xi, xj, (((1,), (1,)), ((), ())),
                            preferred_element_type=jnp.float32)
    d2 = ni_ref[...] + (nj_ref[...] - 2.0 * g)

    acc_ref[pl.ds(ii * b, b), :] = _merge_top4(acc_ref[pl.ds(ii * b, b), :],
                                               d2)

    @pl.when(ii != jj)
    def _col():
        acc_ref[pl.ds(jj * b, b), :] = _merge_top4(
            acc_ref[pl.ds(jj * b, b), :], _col_top4(d2))

    @pl.when(t == nt - 1)
    def _emit():
        out_ref[...] = jnp.maximum(acc_ref[:, K4 - 1:K4], 0.0)


def _radii2(x, norms_col, norms_row, b):
    nb = N // b
    pairs = [(i, j) for i in range(nb) for j in range(i, nb)]
    ii = jnp.asarray([p[0] for p in pairs], dtype=jnp.int32)
    jj = jnp.asarray([p[1] for p in pairs], dtype=jnp.int32)
    grid_spec = pltpu.PrefetchScalarGridSpec(
        num_scalar_prefetch=2,
        grid=(len(pairs),),
        in_specs=[
            pl.BlockSpec((b, D), lambda t, ii, jj: (ii[t], 0)),
            pl.BlockSpec((b, D), lambda t, ii, jj: (jj[t], 0)),
            pl.BlockSpec((b, 1), lambda t, ii, jj: (ii[t], 0)),
            pl.BlockSpec((1, b), lambda t, ii, jj: (0, jj[t])),
        ],
        out_specs=pl.BlockSpec((N, 1), lambda t, ii, jj: (0, 0)),
        scratch_shapes=[pltpu.VMEM((N, K4), jnp.float32)],
    )
    return pl.pallas_call(
        _radii_tri_body,
        grid_spec=grid_spec,
        out_shape=jax.ShapeDtypeStruct((N, 1), jnp.float32),
        compiler_params=pltpu.CompilerParams(
            dimension_semantics=("arbitrary",),
        ),
    )(ii, jj, x, x, norms_col, norms_row)


def _mask_body(f_ref, r_ref, nf_ref, nr_ref, rr_ref, rf_ref, out_ref,
               p_scr, r_scr):
    i = pl.program_id(0)
    j = pl.program_id(1)
    ni_ = pl.num_programs(0)
    nj_ = pl.num_programs(1)
    bm = f_ref.shape[0]
    bn = r_ref.shape[0]

    @pl.when((i == 0) & (j == 0))
    def _init():
        p_scr[...] = jnp.zeros_like(p_scr)
        r_scr[...] = jnp.zeros_like(r_scr)

    f = f_ref[...]
    r = r_ref[...]
    g = jax.lax.dot_general(f, r, (((1,), (1,)), ((), ())),
                            preferred_element_type=jnp.float32)
    d2 = nf_ref[...] + (nr_ref[...] - 2.0 * g)

    hit_p = jnp.max((d2 <= rr_ref[...]).astype(jnp.float32), axis=1,
                    keepdims=True)
    p_scr[pl.ds(i * bm, bm), :] = jnp.maximum(p_scr[pl.ds(i * bm, bm), :],
                                              hit_p)
    hit_r = jnp.max((d2 <= rf_ref[...]).astype(jnp.float32), axis=0,
                    keepdims=True)
    r_scr[:, pl.ds(j * bn, bn)] = jnp.maximum(r_scr[:, pl.ds(j * bn, bn)],
                                              hit_r)

    @pl.when((i == ni_ - 1) & (j == nj_ - 1))
    def _emit():
        out_ref[0, 0] = jnp.sum(p_scr[...]) * (1.0 / N)
        out_ref[0, 1] = jnp.sum(r_scr[...]) * (1.0 / N)


def _masks(fake, real, nf_col, nr_row, radii2_real_row, radii2_fake_col,
           bm, bn):
    grid = (N // bm, N // bn)
    return pl.pallas_call(
        _mask_body,
        grid=grid,
        in_specs=[
            pl.BlockSpec((bm, D), lambda i, j: (i, 0)),
            pl.BlockSpec((bn, D), lambda i, j: (j, 0)),
            pl.BlockSpec((bm, 1), lambda i, j: (i, 0)),
            pl.BlockSpec((1, bn), lambda i, j: (0, j)),
            pl.BlockSpec((1, bn), lambda i, j: (0, j)),
            pl.BlockSpec((bm, 1), lambda i, j: (i, 0)),
        ],
        out_specs=pl.BlockSpec(memory_space=pltpu.SMEM),
        out_shape=jax.ShapeDtypeStruct((1, 2), jnp.float32),
        scratch_shapes=[
            pltpu.VMEM((N, 1), jnp.float32),
            pltpu.VMEM((1, N), jnp.float32),
        ],
        compiler_params=pltpu.CompilerParams(
            dimension_semantics=("arbitrary", "arbitrary"),
        ),
    )(fake, real, nf_col, nr_row, radii2_real_row, radii2_fake_col)


@functools.partial(jax.jit, static_argnames=())
def kernel(real_feats, fake_feats):
    nr = _sq_norms(real_feats)
    nf = _sq_norms(fake_feats)
    nr_row = nr.reshape(1, N)
    nf_row = nf.reshape(1, N)
    real_b = real_feats.astype(jnp.bfloat16)
    fake_b = fake_feats.astype(jnp.bfloat16)
    radii2_real = _radii2(real_b, nr, nr_row, RB)
    radii2_fake = _radii2(fake_b, nf, nf_row, RB)
    out = _masks(fake_b, real_b, nf, nr_row,
                 radii2_real.reshape(1, N), radii2_fake, MB, NB)
    return out.reshape(2)


# speedup vs baseline: 1.6955x; 1.1271x over previous
import functools

import jax
import jax.numpy as jnp
from jax.experimental import pallas as pl
from jax.experimental.pallas import tpu as pltpu

N = 8192
D = 2048
K4 = 4
RB = 1024
MB, NB = 512, 512


def _extract4(work, axis):
    outs = []
    for t in range(K4):
        m = jnp.min(work, axis=axis, keepdims=True)
        outs.append(m)
        if t < K4 - 1:
            work = jnp.where(work == m, jnp.inf, work)
    return outs


def _merge_acc(acc_seg, new4):
    cand = jnp.concatenate([acc_seg, new4], axis=0)
    return jnp.concatenate(_extract4(cand, 0), axis=0)


def _norms_body(x_ref, out_ref):
    x = x_ref[...]
    out_ref[...] = jnp.sum(x * x, axis=1, keepdims=True)


def _sq_norms(x):
    b = min(1024, N)
    return pl.pallas_call(
        _norms_body,
        grid=(N // b,),
        in_specs=[pl.BlockSpec((b, D), lambda i: (i, 0))],
        out_specs=pl.BlockSpec((b, 1), lambda i: (i, 0)),
        out_shape=jax.ShapeDtypeStruct((N, 1), jnp.float32),
    )(x)


def _radii_tri_body(ii_ref, jj_ref, xi_ref, xjm2_ref, ni_ref, nj_ref,
                    nrow_ref, out_ref, acc_ref):
    t = pl.program_id(0)
    nt = pl.num_programs(0)
    b = xi_ref.shape[0]
    ii = ii_ref[t]
    jj = jj_ref[t]

    @pl.when(t == 0)
    def _init():
        acc_ref[...] = jnp.full_like(acc_ref, jnp.inf)

    gm2 = jax.lax.dot_general(xi_ref[...], xjm2_ref[...],
                              (((1,), (1,)), ((), ())),
                              preferred_element_type=jnp.float32)

    u = nj_ref[...] + gm2
    row4 = jnp.transpose(jnp.concatenate(_extract4(u, 1), axis=1))
    acc_ref[:, pl.ds(ii * b, b)] = _merge_acc(acc_ref[:, pl.ds(ii * b, b)],
                                              row4)

    @pl.when(ii != jj)
    def _col():
        v = ni_ref[...] + gm2
        col4 = jnp.concatenate(_extract4(v, 0), axis=0)
        acc_ref[:, pl.ds(jj * b, b)] = _merge_acc(
            acc_ref[:, pl.ds(jj * b, b)], col4)

    @pl.when(t == nt - 1)
    def _emit():
        out_ref[...] = jnp.maximum(acc_ref[K4 - 1:K4, :] + nrow_ref[...],
                                   0.0)


def _radii2_row(x_b, xm2_b, norms_col, norms_row, b):
    nb = N // b
    pairs = [(i, j) for i in range(nb) for j in range(i, nb)]
    ii = jnp.asarray([p[0] for p in pairs], dtype=jnp.int32)
    jj = jnp.asarray([p[1] for p in pairs], dtype=jnp.int32)
    grid_spec = pltpu.PrefetchScalarGridSpec(
        num_scalar_prefetch=2,
        grid=(len(pairs),),
        in_specs=[
            pl.BlockSpec((b, D), lambda t, ii, jj: (ii[t], 0)),
            pl.BlockSpec((b, D), lambda t, ii, jj: (jj[t], 0)),
            pl.BlockSpec((b, 1), lambda t, ii, jj: (ii[t], 0)),
            pl.BlockSpec((1, b), lambda t, ii, jj: (0, jj[t])),
            pl.BlockSpec((1, N), lambda t, ii, jj: (0, 0)),
        ],
        out_specs=pl.BlockSpec((1, N), lambda t, ii, jj: (0, 0)),
        scratch_shapes=[pltpu.VMEM((K4, N), jnp.float32)],
    )
    return pl.pallas_call(
        _radii_tri_body,
        grid_spec=grid_spec,
        out_shape=jax.ShapeDtypeStruct((1, N), jnp.float32),
        compiler_params=pltpu.CompilerParams(
            dimension_semantics=("arbitrary",),
        ),
    )(ii, jj, x_b, xm2_b, norms_col, norms_row, norms_row)


def _mask_body(f_ref, rm2_ref, nf_ref, nr_ref, rr_ref, rf_ref, out_ref,
               p_scr, r_scr):
    i = pl.program_id(0)
    j = pl.program_id(1)
    ni_ = pl.num_programs(0)
    nj_ = pl.num_programs(1)
    bm = f_ref.shape[0]
    bn = rm2_ref.shape[0]

    @pl.when((i == 0) & (j == 0))
    def _init():
        p_scr[...] = jnp.full_like(p_scr, jnp.inf)
        r_scr[...] = jnp.full_like(r_scr, jnp.inf)

    gm2 = jax.lax.dot_general(f_ref[...], rm2_ref[...],
                              (((1,), (1,)), ((), ())),
                              preferred_element_type=jnp.float32)
    u = nr_ref[...] + gm2

    s = u - rr_ref[...]
    mp = jnp.min(s, axis=1, keepdims=True) + nf_ref[...]
    p_scr[pl.ds(i * bm, bm), :] = jnp.minimum(p_scr[pl.ds(i * bm, bm), :],
                                              mp)

    w = nf_ref[...] - rf_ref[...]
    mr = jnp.min(u + w, axis=0, keepdims=True)
    r_scr[:, pl.ds(j * bn, bn)] = jnp.minimum(r_scr[:, pl.ds(j * bn, bn)],
                                              mr)

    @pl.when((i == ni_ - 1) & (j == nj_ - 1))
    def _emit():
        out_ref[0, 0] = jnp.sum(
            jnp.where(p_scr[...] <= 0.0, 1.0, 0.0)) * (1.0 / N)
        out_ref[0, 1] = jnp.sum(
            jnp.where(r_scr[...] <= 0.0, 1.0, 0.0)) * (1.0 / N)


def _masks(fake_b, realm2_b, nf_col, nr_row, radii2_real_row,
           radii2_fake_col, bm, bn):
    grid = (N // bm, N // bn)
    return pl.pallas_call(
        _mask_body,
        grid=grid,
        in_specs=[
            pl.BlockSpec((bm, D), lambda i, j: (i, 0)),
            pl.BlockSpec((bn, D), lambda i, j: (j, 0)),
            pl.BlockSpec((bm, 1), lambda i, j: (i, 0)),
            pl.BlockSpec((1, bn), lambda i, j: (0, j)),
            pl.BlockSpec((1, bn), lambda i, j: (0, j)),
            pl.BlockSpec((bm, 1), lambda i, j: (i, 0)),
        ],
        out_specs=pl.BlockSpec(memory_space=pltpu.SMEM),
        out_shape=jax.ShapeDtypeStruct((1, 2), jnp.float32),
        scratch_shapes=[
            pltpu.VMEM((N, 1), jnp.float32),
            pltpu.VMEM((1, N), jnp.float32),
        ],
        compiler_params=pltpu.CompilerParams(
            dimension_semantics=("arbitrary", "arbitrary"),
        ),
    )(fake_b, realm2_b, nf_col, nr_row, radii2_real_row, radii2_fake_col)


@functools.partial(jax.jit, static_argnames=())
def kernel(real_feats, fake_feats):
    nr = _sq_norms(real_feats)
    nf = _sq_norms(fake_feats)
    nr_row = nr.reshape(1, N)
    nf_row = nf.reshape(1, N)
    real_b = real_feats.astype(jnp.bfloat16)
    fake_b = fake_feats.astype(jnp.bfloat16)
    realm2_b = (real_feats * -2.0).astype(jnp.bfloat16)
    fakem2_b = (fake_feats * -2.0).astype(jnp.bfloat16)
    rr_row = _radii2_row(real_b, realm2_b, nr, nr_row, RB)
    rf_row = _radii2_row(fake_b, fakem2_b, nf, nf_row, RB)
    out = _masks(fake_b, realm2_b, nf, nr_row,
                 rr_row, rf_row.reshape(N, 1), MB, NB)
    return out.reshape(2)


# speedup vs baseline: 1.8245x; 1.0761x over previous
import functools

import jax
import jax.numpy as jnp
from jax.experimental import pallas as pl
from jax.experimental.pallas import tpu as pltpu

N = 8192
D = 2048
K4 = 4
RB = 1024
MB, NB = 512, 1024


def _extract4(work, axis):
    outs = []
    for t in range(K4):
        m = jnp.min(work, axis=axis, keepdims=True)
        outs.append(m)
        if t < K4 - 1:
            work = jnp.where(work == m, jnp.inf, work)
    return outs


def _merge_acc(acc_seg, new4):
    cand = jnp.concatenate([acc_seg, new4], axis=0)
    return jnp.concatenate(_extract4(cand, 0), axis=0)


def _norms_body(x_ref, out_ref):
    x = x_ref[...]
    out_ref[...] = jnp.sum(x * x, axis=1, keepdims=True)


def _sq_norms(x):
    b = min(1024, N)
    return pl.pallas_call(
        _norms_body,
        grid=(N // b,),
        in_specs=[pl.BlockSpec((b, D), lambda i: (i, 0))],
        out_specs=pl.BlockSpec((b, 1), lambda i: (i, 0)),
        out_shape=jax.ShapeDtypeStruct((N, 1), jnp.float32),
    )(x)


def _radii_tri_body(ii_ref, jj_ref, xi_ref, xjm2_ref, ni_ref, nj_ref,
                    nrow_ref, out_ref, acc_ref):
    t = pl.program_id(0)
    nt = pl.num_programs(0)
    b = xi_ref.shape[0]
    ii = ii_ref[t]
    jj = jj_ref[t]

    @pl.when(t == 0)
    def _init():
        acc_ref[...] = jnp.full_like(acc_ref, jnp.inf)

    gm2 = jax.lax.dot_general(xi_ref[...], xjm2_ref[...],
                              (((1,), (1,)), ((), ())),
                              preferred_element_type=jnp.float32)

    u = nj_ref[...] + gm2
    row4 = jnp.transpose(jnp.concatenate(_extract4(u, 1), axis=1))
    acc_ref[:, pl.ds(ii * b, b)] = _merge_acc(acc_ref[:, pl.ds(ii * b, b)],
                                              row4)

    @pl.when(ii != jj)
    def _col():
        v = ni_ref[...] + gm2
        col4 = jnp.concatenate(_extract4(v, 0), axis=0)
        acc_ref[:, pl.ds(jj * b, b)] = _merge_acc(
            acc_ref[:, pl.ds(jj * b, b)], col4)

    @pl.when(t == nt - 1)
    def _emit():
        out_ref[...] = jnp.maximum(acc_ref[K4 - 1:K4, :] + nrow_ref[...],
                                   0.0)


def _radii2_row(x_b, xm2_b, norms_col, norms_row, b):
    nb = N // b
    pairs = [(i, j) for i in range(nb) for j in range(i, nb)]
    ii = jnp.asarray([p[0] for p in pairs], dtype=jnp.int32)
    jj = jnp.asarray([p[1] for p in pairs], dtype=jnp.int32)
    grid_spec = pltpu.PrefetchScalarGridSpec(
        num_scalar_prefetch=2,
        grid=(len(pairs),),
        in_specs=[
            pl.BlockSpec((b, D), lambda t, ii, jj: (ii[t], 0)),
            pl.BlockSpec((b, D), lambda t, ii, jj: (jj[t], 0)),
            pl.BlockSpec((b, 1), lambda t, ii, jj: (ii[t], 0)),
            pl.BlockSpec((1, b), lambda t, ii, jj: (0, jj[t])),
            pl.BlockSpec((1, N), lambda t, ii, jj: (0, 0)),
        ],
        out_specs=pl.BlockSpec((1, N), lambda t, ii, jj: (0, 0)),
        scratch_shapes=[pltpu.VMEM((K4, N), jnp.float32)],
    )
    return pl.pallas_call(
        _radii_tri_body,
        grid_spec=grid_spec,
        out_shape=jax.ShapeDtypeStruct((1, N), jnp.float32),
        compiler_params=pltpu.CompilerParams(
            dimension_semantics=("arbitrary",),
        ),
    )(ii, jj, x_b, xm2_b, norms_col, norms_row, norms_row)


def _mask_body(f_ref, rm2_ref, nf_ref, nr_ref, rr_ref, rf_ref, out_ref,
               p_scr, r_scr):
    i = pl.program_id(0)
    j = pl.program_id(1)
    ni_ = pl.num_programs(0)
    nj_ = pl.num_programs(1)
    bm = f_ref.shape[0]
    bn = rm2_ref.shape[0]

    @pl.when((i == 0) & (j == 0))
    def _init():
        p_scr[...] = jnp.full_like(p_scr, jnp.inf)
        r_scr[...] = jnp.full_like(r_scr, jnp.inf)

    gm2 = jax.lax.dot_general(f_ref[...], rm2_ref[...],
                              (((1,), (1,)), ((), ())),
                              preferred_element_type=jnp.float32)

    s = gm2 + (nr_ref[...] - rr_ref[...])
    mp = jnp.min(s, axis=1, keepdims=True) + nf_ref[...]
    p_scr[pl.ds(i * bm, bm), :] = jnp.minimum(p_scr[pl.ds(i * bm, bm), :],
                                              mp)

    t = gm2 + (nf_ref[...] - rf_ref[...])
    mr = jnp.min(t, axis=0, keepdims=True) + nr_ref[...]
    r_scr[:, pl.ds(j * bn, bn)] = jnp.minimum(r_scr[:, pl.ds(j * bn, bn)],
                                              mr)

    @pl.when((i == ni_ - 1) & (j == nj_ - 1))
    def _emit():
        out_ref[0, 0] = jnp.sum(
            jnp.where(p_scr[...] <= 0.0, 1.0, 0.0)) * (1.0 / N)
        out_ref[0, 1] = jnp.sum(
            jnp.where(r_scr[...] <= 0.0, 1.0, 0.0)) * (1.0 / N)


def _masks(fake_b, realm2_b, nf_col, nr_row, radii2_real_row,
           radii2_fake_col, bm, bn):
    grid = (N // bm, N // bn)
    return pl.pallas_call(
        _mask_body,
        grid=grid,
        in_specs=[
            pl.BlockSpec((bm, D), lambda i, j: (i, 0)),
            pl.BlockSpec((bn, D), lambda i, j: (j, 0)),
            pl.BlockSpec((bm, 1), lambda i, j: (i, 0)),
            pl.BlockSpec((1, bn), lambda i, j: (0, j)),
            pl.BlockSpec((1, bn), lambda i, j: (0, j)),
            pl.BlockSpec((bm, 1), lambda i, j: (i, 0)),
        ],
        out_specs=pl.BlockSpec(memory_space=pltpu.SMEM),
        out_shape=jax.ShapeDtypeStruct((1, 2), jnp.float32),
        scratch_shapes=[
            pltpu.VMEM((N, 1), jnp.float32),
            pltpu.VMEM((1, N), jnp.float32),
        ],
        compiler_params=pltpu.CompilerParams(
            dimension_semantics=("arbitrary", "arbitrary"),
        ),
    )(fake_b, realm2_b, nf_col, nr_row, radii2_real_row, radii2_fake_col)


@functools.partial(jax.jit, static_argnames=())
def kernel(real_feats, fake_feats):
    nr = _sq_norms(real_feats)
    nf = _sq_norms(fake_feats)
    nr_row = nr.reshape(1, N)
    nf_row = nf.reshape(1, N)
    real_b = real_feats.astype(jnp.bfloat16)
    fake_b = fake_feats.astype(jnp.bfloat16)
    realm2_b = (real_feats * -2.0).astype(jnp.bfloat16)
    fakem2_b = (fake_feats * -2.0).astype(jnp.bfloat16)
    rr_row = _radii2_row(real_b, realm2_b, nr, nr_row, RB)
    rf_row = _radii2_row(fake_b, fakem2_b, nf, nf_row, RB)
    out = _masks(fake_b, realm2_b, nf, nr_row,
                 rr_row, rf_row.reshape(N, 1), MB, NB)
    return out.reshape(2)


# speedup vs baseline: 1.8954x; 1.0388x over previous
import functools

import jax
import jax.numpy as jnp
from jax.experimental import pallas as pl
from jax.experimental.pallas import tpu as pltpu

N = 8192
D = 2048
K4 = 4
RB = 1024
MB, NB = 1024, 1024


def _extract4(work, axis):
    outs = []
    for t in range(K4):
        m = jnp.min(work, axis=axis, keepdims=True)
        outs.append(m)
        if t < K4 - 1:
            work = jnp.where(work == m, jnp.inf, work)
    return outs


def _merge_acc(acc_seg, new4):
    cand = jnp.concatenate([acc_seg, new4], axis=0)
    return jnp.concatenate(_extract4(cand, 0), axis=0)


def _norms_body(x_ref, out_ref):
    x = x_ref[...]
    out_ref[...] = jnp.sum(x * x, axis=1, keepdims=True)


def _sq_norms(x):
    b = min(1024, N)
    return pl.pallas_call(
        _norms_body,
        grid=(N // b,),
        in_specs=[pl.BlockSpec((b, D), lambda i: (i, 0))],
        out_specs=pl.BlockSpec((b, 1), lambda i: (i, 0)),
        out_shape=jax.ShapeDtypeStruct((N, 1), jnp.float32),
    )(x)


def _radii_tri_body(ii_ref, jj_ref, xi_ref, xjm2_ref, ni_ref, nj_ref,
                    nrow_ref, out_ref, acc_ref):
    t = pl.program_id(0)
    nt = pl.num_programs(0)
    b = xi_ref.shape[0]
    ii = ii_ref[t]
    jj = jj_ref[t]

    @pl.when(t == 0)
    def _init():
        acc_ref[...] = jnp.full_like(acc_ref, jnp.inf)

    gm2 = jax.lax.dot_general(xi_ref[...], xjm2_ref[...],
                              (((1,), (1,)), ((), ())),
                              preferred_element_type=jnp.float32)

    u = nj_ref[...] + gm2
    row4 = jnp.transpose(jnp.concatenate(_extract4(u, 1), axis=1))
    acc_ref[:, pl.ds(ii * b, b)] = _merge_acc(acc_ref[:, pl.ds(ii * b, b)],
                                              row4)

    @pl.when(ii != jj)
    def _col():
        v = ni_ref[...] + gm2
        col4 = jnp.concatenate(_extract4(v, 0), axis=0)
        acc_ref[:, pl.ds(jj * b, b)] = _merge_acc(
            acc_ref[:, pl.ds(jj * b, b)], col4)

    @pl.when(t == nt - 1)
    def _emit():
        out_ref[...] = jnp.maximum(acc_ref[K4 - 1:K4, :] + nrow_ref[...],
                                   0.0)


def _radii2_row(x_b, xm2_b, norms_col, norms_row, b):
    nb = N // b
    pairs = [(i, j) for i in range(nb) for j in range(i, nb)]
    ii = jnp.asarray([p[0] for p in pairs], dtype=jnp.int32)
    jj = jnp.asarray([p[1] for p in pairs], dtype=jnp.int32)
    grid_spec = pltpu.PrefetchScalarGridSpec(
        num_scalar_prefetch=2,
        grid=(len(pairs),),
        in_specs=[
            pl.BlockSpec((b, D), lambda t, ii, jj: (ii[t], 0)),
            pl.BlockSpec((b, D), lambda t, ii, jj: (jj[t], 0)),
            pl.BlockSpec((b, 1), lambda t, ii, jj: (ii[t], 0)),
            pl.BlockSpec((1, b), lambda t, ii, jj: (0, jj[t])),
            pl.BlockSpec((1, N), lambda t, ii, jj: (0, 0)),
        ],
        out_specs=pl.BlockSpec((1, N), lambda t, ii, jj: (0, 0)),
        scratch_shapes=[pltpu.VMEM((K4, N), jnp.float32)],
    )
    return pl.pallas_call(
        _radii_tri_body,
        grid_spec=grid_spec,
        out_shape=jax.ShapeDtypeStruct((1, N), jnp.float32),
        compiler_params=pltpu.CompilerParams(
            dimension_semantics=("arbitrary",),
        ),
    )(ii, jj, x_b, xm2_b, norms_col, norms_row, norms_row)


def _mask_body(f_ref, rm2_ref, nf_ref, nr_ref, rr_ref, rf_ref, out_ref,
               p_scr, r_scr):
    i = pl.program_id(0)
    j = pl.program_id(1)
    ni_ = pl.num_programs(0)
    nj_ = pl.num_programs(1)
    bm = f_ref.shape[0]
    bn = rm2_ref.shape[0]

    @pl.when((i == 0) & (j == 0))
    def _init():
        p_scr[...] = jnp.full_like(p_scr, jnp.inf)
        r_scr[...] = jnp.full_like(r_scr, jnp.inf)

    gm2 = jax.lax.dot_general(f_ref[...], rm2_ref[...],
                              (((1,), (1,)), ((), ())),
                              preferred_element_type=jnp.float32)

    s = gm2 + (nr_ref[...] - rr_ref[...])
    mp = jnp.min(s, axis=1, keepdims=True) + nf_ref[...]
    p_scr[pl.ds(i * bm, bm), :] = jnp.minimum(p_scr[pl.ds(i * bm, bm), :],
                                              mp)

    t = gm2 + (nf_ref[...] - rf_ref[...])
    mr = jnp.min(t, axis=0, keepdims=True) + nr_ref[...]
    r_scr[:, pl.ds(j * bn, bn)] = jnp.minimum(r_scr[:, pl.ds(j * bn, bn)],
                                              mr)

    @pl.when((i == ni_ - 1) & (j == nj_ - 1))
    def _emit():
        out_ref[0, 0] = jnp.sum(
            jnp.where(p_scr[...] <= 0.0, 1.0, 0.0)) * (1.0 / N)
        out_ref[0, 1] = jnp.sum(
            jnp.where(r_scr[...] <= 0.0, 1.0, 0.0)) * (1.0 / N)


def _masks(fake_b, realm2_b, nf_col, nr_row, radii2_real_row,
           radii2_fake_col, bm, bn):
    grid = (N // bm, N // bn)
    return pl.pallas_call(
        _mask_body,
        grid=grid,
        in_specs=[
            pl.BlockSpec((bm, D), lambda i, j: (i, 0)),
            pl.BlockSpec((bn, D), lambda i, j: (j, 0)),
            pl.BlockSpec((bm, 1), lambda i, j: (i, 0)),
            pl.BlockSpec((1, bn), lambda i, j: (0, j)),
            pl.BlockSpec((1, bn), lambda i, j: (0, j)),
            pl.BlockSpec((bm, 1), lambda i, j: (i, 0)),
        ],
        out_specs=pl.BlockSpec(memory_space=pltpu.SMEM),
        out_shape=jax.ShapeDtypeStruct((1, 2), jnp.float32),
        scratch_shapes=[
            pltpu.VMEM((N, 1), jnp.float32),
            pltpu.VMEM((1, N), jnp.float32),
        ],
        compiler_params=pltpu.CompilerParams(
            dimension_semantics=("arbitrary", "arbitrary"),
        ),
    )(fake_b, realm2_b, nf_col, nr_row, radii2_real_row, radii2_fake_col)


@functools.partial(jax.jit, static_argnames=())
def kernel(real_feats, fake_feats):
    nr = _sq_norms(real_feats)
    nf = _sq_norms(fake_feats)
    nr_row = nr.reshape(1, N)
    nf_row = nf.reshape(1, N)
    real_b = real_feats.astype(jnp.bfloat16)
    fake_b = fake_feats.astype(jnp.bfloat16)
    realm2_b = (real_feats * -2.0).astype(jnp.bfloat16)
    fakem2_b = (fake_feats * -2.0).astype(jnp.bfloat16)
    rr_row = _radii2_row(real_b, realm2_b, nr, nr_row, RB)
    rf_row = _radii2_row(fake_b, fakem2_b, nf, nf_row, RB)
    out = _masks(fake_b, realm2_b, nf, nr_row,
                 rr_row, rf_row.reshape(N, 1), MB, NB)
    return out.reshape(2)
